# baseline (speedup 1.0000x reference)
import jax
import jax.numpy as jnp
from jax import lax
from jax.experimental import pallas as pl
from jax.experimental.pallas import tpu as pltpu

_B, _S, _DM, _DCQ = 2, 2048, 2048, 1536
_H, _D, _ROPE, _TOPK = 4, 128, 64, 512
_SB = 128
_TB = 256
_NEG = -1e9
_DSCALE = _D ** -0.5


def _rope_apply(v, c, a, bm):
    r = jnp.concatenate([v[..., -1:], v[..., :-1]], axis=-1)
    l = jnp.concatenate([v[..., 1:], v[..., :1]], axis=-1)
    return v * c + r * a + l * bm


def _keys_kernel(x_ref, wdk_ref, bdk_ref, g_ref, b_ref, cde_ref, wwp_ref,
                 bwp_ref, k_out_ref, wt_out_ref):
    x = x_ref[0]
    h = lax.dot_general(x, wdk_ref[...], (((1,), (0,)), ((), ())),
                        preferred_element_type=jnp.float32) + bdk_ref[...]
    mu = jnp.mean(h, axis=-1, keepdims=True)
    var = jnp.mean((h - mu) ** 2, axis=-1, keepdims=True)
    h = (h - mu) * lax.rsqrt(var + 1e-6) * g_ref[...] + b_ref[...]
    k_out_ref[0] = _rope_apply(h, cde_ref[0], cde_ref[1], cde_ref[2])
    wt = lax.dot_general(wwp_ref[...], x, (((0,), (1,)), ((), ())),
                         preferred_element_type=jnp.float32)
    wt_out_ref[0] = (wt + bwp_ref[...]) * 0.5


def _lexgt(av, ai, bv, bi):
    return (av > bv) | ((av == bv) & (ai < bi))


def _main_kernel(cq_ref, wuq_ref, buq_ref, cde_ref, k_ref, wt_ref, out_ref):
    sb = pl.program_id(1)
    q = lax.dot_general(cq_ref[0], wuq_ref[...], (((1,), (0,)), ((), ())),
                        preferred_element_type=jnp.float32) + buq_ref[...]
    q3 = q.reshape(_SB, _H, _D)
    q3 = _rope_apply(q3, cde_ref[0][:, None, :], cde_ref[1][:, None, :],
                     cde_ref[2][:, None, :])
    k = k_ref[0]
    p = []
    for h in range(_H):
        sc = lax.dot_general(k, q3[:, h, :], (((1,), (1,)), ((), ())),
                             preferred_element_type=jnp.float32)
        scb = jnp.maximum(sc, 0.0).astype(jnp.bfloat16).astype(jnp.float32)
        wb = wt_ref[0, h:h + 1, :].astype(jnp.bfloat16).astype(jnp.float32)
        p.append(scb * wb)
    acc = (p[0] + p[1]) + (p[2] + p[3])
    t_idx = lax.broadcasted_iota(jnp.int32, (_S, _SB), 0)
    s_glob = sb * _SB + lax.broadcasted_iota(jnp.int32, (_S, _SB), 1)
    vals = acc * _DSCALE + jnp.where(t_idx > s_glob, _NEG, 0.0)
    idx = t_idx
    if out_ref.dtype == jnp.float32:
        out_ref[0, 0] = vals
        return

    kk = 2
    while kk <= _S:
        j = kk // 2
        while j >= 1:
            m = _S // (2 * j)
            vr = vals.reshape(m, 2, j, _SB)
            ir = idx.reshape(m, 2, j, _SB)
            av, bv = vr[:, 0], vr[:, 1]
            ai, bi = ir[:, 0], ir[:, 1]
            gbit = kk // (2 * j)
            dird = (lax.broadcasted_iota(jnp.int32, (m, j, _SB), 0)
                    & gbit) == 0
            cond = _lexgt(av, ai, bv, bi) == dird
            nav = jnp.where(cond, av, bv)
            nbv = jnp.where(cond, bv, av)
            nai = jnp.where(cond, ai, bi)
            nbi = jnp.where(cond, bi, ai)
            vals = jnp.concatenate([nav[:, None], nbv[:, None]],
                                   axis=1).reshape(_S, _SB)
            idx = jnp.concatenate([nai[:, None], nbi[:, None]],
                                  axis=1).reshape(_S, _SB)
            j //= 2
        kk *= 2
    out_ref[0, 0] = idx[:_TOPK, :]


_DEBUG_SCORES = False
_DEBUG_XLA_KEYS = True


def kernel(x, c_Q, mask, freq_cis, pos, W_UQI, b_UQI, W_DKI, b_DKI,
           ln_g, ln_b, W_wp, b_wp):
    del mask, pos
    f32 = jnp.float32

    wuq = W_UQI
    buq = b_UQI.reshape(1, _H * _D)
    wdk = W_DKI
    bdk = b_DKI.reshape(1, _D)
    g2 = ln_g.reshape(1, _D)
    b2 = ln_b.reshape(1, _D)

    wwp = jnp.zeros((_DM, 8), f32).at[:, :_H].set(W_wp)
    bwp = jnp.broadcast_to(
        (jnp.zeros((8,), f32).at[:_H].set(b_wp))[:, None], (8, _S))

    cos = freq_cis[..., 0]
    sin = freq_cis[..., 1]
    zer = jnp.zeros((_S, _ROPE // 2), f32)
    one = jnp.ones((_S, _ROPE), f32)

    def ilv(a, b):
        return jnp.stack([a, b], axis=-1).reshape(_S, _ROPE)

    ctab = jnp.concatenate([ilv(cos, cos), one], axis=1)
    atab = jnp.concatenate([ilv(zer, sin), 0 * one], axis=1)
    btab = jnp.concatenate([ilv(-sin, zer), 0 * one], axis=1)
    cde = jnp.stack([ctab, atab, btab])

    if _DEBUG_XLA_KEYS:
        def _ln(h, g, b):
            mu = jnp.mean(h, axis=-1, keepdims=True)
            var = jnp.var(h, axis=-1, keepdims=True)
            return (h - mu) * lax.rsqrt(var + 1e-6) * g + b

        def _rope_ref(v, c, s):
            v2 = v.reshape(v.shape[:-1] + (v.shape[-1] // 2, 2))
            x1 = v2[..., 0]
            x2 = v2[..., 1]
            return jnp.stack([x1 * c - x2 * s, x1 * s + x2 * c],
                             axis=-1).reshape(v.shape)

        k_I0 = _ln(x @ W_DKI + b_DKI, ln_g, ln_b)
        kr = _rope_ref(k_I0[..., :_ROPE], cos[None], sin[None])
        k_i = jnp.concatenate([kr, k_I0[..., _ROPE:]], axis=-1)
        w_full = (x @ W_wp + b_wp) * 0.5
        w_t = jnp.zeros((_B, 8, _S), f32).at[:, :_H, :].set(
            w_full.transpose(0, 2, 1))
        return _call_main(c_Q, wuq, buq, cde, k_i, w_t)

    k_i, w_t = pl.pallas_call(
        _keys_kernel,
        grid=(_B, _S // _TB),
        in_specs=[
            pl.BlockSpec((1, _TB, _DM), lambda b, t: (b, t, 0)),
            pl.BlockSpec((_DM, _D), lambda b, t: (0, 0)),
            pl.BlockSpec((1, _D), lambda b, t: (0, 0)),
            pl.BlockSpec((1, _D), lambda b, t: (0, 0)),
            pl.BlockSpec((1, _D), lambda b, t: (0, 0)),
            pl.BlockSpec((3, _TB, _D), lambda b, t: (0, t, 0)),
            pl.BlockSpec((_DM, 8), lambda b, t: (0, 0)),
            pl.BlockSpec((8, _TB), lambda b, t: (0, t)),
        ],
        out_specs=[
            pl.BlockSpec((1, _TB, _D), lambda b, t: (b, t, 0)),
            pl.BlockSpec((1, 8, _TB), lambda b, t: (b, 0, t)),
        ],
        out_shape=[
            jax.ShapeDtypeStruct((_B, _S, _D), f32),
            jax.ShapeDtypeStruct((_B, 8, _S), f32),
        ],
    )(x, wdk, bdk, g2, b2, cde, wwp, bwp)

    return _call_main(c_Q, wuq, buq, cde, k_i, w_t)


def _call_main(c_Q, wuq, buq, cde, k_i, w_t):
    nsb = _S // _SB
    if _DEBUG_SCORES:
        out_spec = pl.BlockSpec((1, 1, _S, _SB), lambda b, s: (b, s, 0, 0))
        out_shape = jax.ShapeDtypeStruct((_B, nsb, _S, _SB), jnp.float32)
    else:
        out_spec = pl.BlockSpec((1, 1, _TOPK, _SB), lambda b, s: (b, s, 0, 0))
        out_shape = jax.ShapeDtypeStruct((_B, nsb, _TOPK, _SB), jnp.int32)
    out = pl.pallas_call(
        _main_kernel,
        grid=(_B, nsb),
        in_specs=[
            pl.BlockSpec((1, _SB, _DCQ), lambda b, s: (b, s, 0)),
            pl.BlockSpec((_DCQ, _H * _D), lambda b, s: (0, 0)),
            pl.BlockSpec((1, _H * _D), lambda b, s: (0, 0)),
            pl.BlockSpec((3, _SB, _D), lambda b, s: (0, s, 0)),
            pl.BlockSpec((1, _S, _D), lambda b, s: (b, 0, 0)),
            pl.BlockSpec((1, 8, _SB), lambda b, s: (b, 0, s)),
        ],
        out_specs=out_spec,
        out_shape=out_shape,
    )(c_Q, wuq, buq, cde, k_i, w_t)

    if _DEBUG_SCORES:
        sc = out.transpose(0, 1, 3, 2).reshape(_B, _S, _S)
        _, ti = lax.top_k(sc, _TOPK)
        return ti
    return out.transpose(0, 1, 3, 2).reshape(_B, _S, _TOPK)


# speedup vs baseline: 2.1125x; 2.1125x over previous
import jax
import jax.numpy as jnp
from jax import lax
from jax.experimental import pallas as pl
from jax.experimental.pallas import tpu as pltpu

_B, _S, _DM, _DCQ = 2, 2048, 2048, 1536
_H, _D, _ROPE, _TOPK = 4, 128, 64, 512
_SB = 128
_TB = 256
_NEG = -1e9
_DSCALE = _D ** -0.5


def _rope_apply(v, c, a, bm):
    r = jnp.concatenate([v[..., -1:], v[..., :-1]], axis=-1)
    l = jnp.concatenate([v[..., 1:], v[..., :1]], axis=-1)
    return v * c + r * a + l * bm


def _keys_kernel(x_ref, wdk_ref, bdk_ref, g_ref, b_ref, cde_ref, wwp_ref,
                 bwp_ref, k_out_ref, w_out_ref):
    x = x_ref[0]
    h = lax.dot_general(x, wdk_ref[...], (((1,), (0,)), ((), ())),
                        preferred_element_type=jnp.float32) + bdk_ref[...]
    mu = jnp.mean(h, axis=-1, keepdims=True)
    var = jnp.mean((h - mu) ** 2, axis=-1, keepdims=True)
    h = (h - mu) * lax.rsqrt(var + 1e-6) * g_ref[...] + b_ref[...]
    k_out_ref[0] = _rope_apply(h, cde_ref[0], cde_ref[1], cde_ref[2])
    w = lax.dot_general(x, wwp_ref[...], (((1,), (0,)), ((), ())),
                        preferred_element_type=jnp.float32)
    w_out_ref[0] = (w + bwp_ref[...]) * 0.5


def _lexgt(av, ai, bv, bi):
    return (av > bv) | ((av == bv) & (ai < bi))


def _roll_l(x, j):
    return jnp.concatenate([x[:, j:], x[:, :j]], axis=1)


def _roll_r(x, j):
    return jnp.concatenate([x[:, -j:], x[:, :-j]], axis=1)


def _main_kernel(cq_ref, wuq_ref, buq_ref, cde_ref, k_ref, w_ref, out_ref):
    sb = pl.program_id(1)
    q = lax.dot_general(cq_ref[0], wuq_ref[...], (((1,), (0,)), ((), ())),
                        preferred_element_type=jnp.float32) + buq_ref[...]
    q3 = q.reshape(_SB, _H, _D)
    q3 = _rope_apply(q3, cde_ref[0][:, None, :], cde_ref[1][:, None, :],
                     cde_ref[2][:, None, :])
    k = k_ref[0]
    p = []
    for h in range(_H):
        sc = lax.dot_general(q3[:, h, :], k, (((1,), (1,)), ((), ())),
                             preferred_element_type=jnp.float32)
        scb = jnp.maximum(sc, 0.0).astype(jnp.bfloat16).astype(jnp.float32)
        wb = w_ref[0][:, h:h + 1].astype(jnp.bfloat16).astype(jnp.float32)
        p.append(scb * wb)
    acc = (p[0] + p[1]) + (p[2] + p[3])
    lane = lax.broadcasted_iota(jnp.int32, (_SB, _S), 1)
    s_glob = sb * _SB + lax.broadcasted_iota(jnp.int32, (_SB, _S), 0)
    vals = acc * _DSCALE + jnp.where(lane > s_glob, _NEG, 0.0)
    idx = lane

    def stage(vals, idx, lane, kk, j, width):
        pv = jnp.where((lane & j) == 0, _roll_l(vals, j), _roll_r(vals, j))
        pi = jnp.where((lane & j) == 0, _roll_l(idx, j), _roll_r(idx, j))
        lgt = _lexgt(vals, idx, pv, pi)
        dird = (lane & kk) == 0
        keep = (lgt == dird) == ((lane & j) == 0)
        return jnp.where(keep, vals, pv), jnp.where(keep, idx, pi)

    kk = 2
    while kk <= _S:
        j = kk // 2
        while j >= 1:
            vals, idx = stage(vals, idx, lane, kk, j, _S)
            j //= 2
        kk *= 2
    out_ref[0, 0] = idx[:, :_TOPK]


_DEBUG_XLA_KEYS = False


def kernel(x, c_Q, mask, freq_cis, pos, W_UQI, b_UQI, W_DKI, b_DKI,
           ln_g, ln_b, W_wp, b_wp):
    del mask, pos
    f32 = jnp.float32

    wuq = W_UQI
    buq = b_UQI.reshape(1, _H * _D)

    wwp = jnp.zeros((_DM, 128), f32).at[:, :_H].set(W_wp)
    bwp = jnp.zeros((1, 128), f32).at[0, :_H].set(b_wp)

    cos = freq_cis[..., 0]
    sin = freq_cis[..., 1]
    zer = jnp.zeros((_S, _ROPE // 2), f32)
    one = jnp.ones((_S, _ROPE), f32)

    def ilv(a, b):
        return jnp.stack([a, b], axis=-1).reshape(_S, _ROPE)

    ctab = jnp.concatenate([ilv(cos, cos), one], axis=1)
    atab = jnp.concatenate([ilv(zer, sin), 0 * one], axis=1)
    btab = jnp.concatenate([ilv(-sin, zer), 0 * one], axis=1)
    cde = jnp.stack([ctab, atab, btab])

    if _DEBUG_XLA_KEYS:
        def _rope_ref(v, c, s):
            v2 = v.reshape(v.shape[:-1] + (v.shape[-1] // 2, 2))
            x1 = v2[..., 0]
            x2 = v2[..., 1]
            return jnp.stack([x1 * c - x2 * s, x1 * s + x2 * c],
                             axis=-1).reshape(v.shape)

        h0 = x @ W_DKI + b_DKI
        mu = jnp.mean(h0, axis=-1, keepdims=True)
        var = jnp.var(h0, axis=-1, keepdims=True)
        k_I0 = (h0 - mu) * lax.rsqrt(var + 1e-6) * ln_g + ln_b
        kr = _rope_ref(k_I0[..., :_ROPE], cos[None], sin[None])
        k_i = jnp.concatenate([kr, k_I0[..., _ROPE:]], axis=-1)
        w_full = (x @ W_wp + b_wp) * 0.5
        w_i = jnp.zeros((_B, _S, 128), f32).at[:, :, :_H].set(w_full)
        return _call_main(c_Q, wuq, buq, cde, k_i, w_i)

    k_i, w_i = pl.pallas_call(
        _keys_kernel,
        grid=(_B, _S // _TB),
        in_specs=[
            pl.BlockSpec((1, _TB, _DM), lambda b, t: (b, t, 0)),
            pl.BlockSpec((_DM, _D), lambda b, t: (0, 0)),
            pl.BlockSpec((1, _D), lambda b, t: (0, 0)),
            pl.BlockSpec((1, _D), lambda b, t: (0, 0)),
            pl.BlockSpec((1, _D), lambda b, t: (0, 0)),
            pl.BlockSpec((3, _TB, _D), lambda b, t: (0, t, 0)),
            pl.BlockSpec((_DM, 128), lambda b, t: (0, 0)),
            pl.BlockSpec((1, 128), lambda b, t: (0, 0)),
        ],
        out_specs=[
            pl.BlockSpec((1, _TB, _D), lambda b, t: (b, t, 0)),
            pl.BlockSpec((1, _TB, 128), lambda b, t: (b, t, 0)),
        ],
        out_shape=[
            jax.ShapeDtypeStruct((_B, _S, _D), f32),
            jax.ShapeDtypeStruct((_B, _S, 128), f32),
        ],
    )(x, W_DKI, b_DKI.reshape(1, _D), ln_g.reshape(1, _D),
      ln_b.reshape(1, _D), cde, wwp, bwp)

    return _call_main(c_Q, wuq, buq, cde, k_i, w_i)


def _call_main(c_Q, wuq, buq, cde, k_i, w_i):
    nsb = _S // _SB
    out = pl.pallas_call(
        _main_kernel,
        grid=(_B, nsb),
        in_specs=[
            pl.BlockSpec((1, _SB, _DCQ), lambda b, s: (b, s, 0)),
            pl.BlockSpec((_DCQ, _H * _D), lambda b, s: (0, 0)),
            pl.BlockSpec((1, _H * _D), lambda b, s: (0, 0)),
            pl.BlockSpec((3, _SB, _D), lambda b, s: (0, s, 0)),
            pl.BlockSpec((1, _S, _D), lambda b, s: (b, 0, 0)),
            pl.BlockSpec((1, _SB, 128), lambda b, s: (b, s, 0)),
        ],
        out_specs=pl.BlockSpec((1, 1, _SB, _TOPK), lambda b, s: (b, s, 0, 0)),
        out_shape=jax.ShapeDtypeStruct((_B, nsb, _SB, _TOPK), jnp.int32),
    )(c_Q, wuq, buq, cde, k_i, w_i)
    return out.reshape(_B, _S, _TOPK)
